# baseline (speedup 1.0000x reference)
import functools

import jax
import jax.numpy as jnp
from jax import lax
from jax.experimental import pallas as pl
from jax.experimental.pallas import tpu as pltpu
from jax.experimental.pallas import tpu_sc as plsc

CHUNK = 128
S = 2
SUPER = S * CHUNK
NBUF = 4


@functools.cache
def _make_gather(H, Bb, D):
    B = H * Bb
    info = plsc.get_sparse_core_info()
    NC, NS = info.num_cores, info.num_subcores
    NW = NC * NS
    assert B % (NW * SUPER) == 0 and Bb % SUPER == 0
    b_per_w = B // NW
    n_chunks = b_per_w // CHUNK
    n_super = b_per_w // SUPER
    nbuf = max(n for n in (NBUF, 2, 1) if n_super % n == 0)

    mesh = plsc.VectorSubcoreMesh(core_axis_name="c", subcore_axis_name="s")

    @functools.partial(
        pl.kernel,
        mesh=mesh,
        out_type=jax.ShapeDtypeStruct((H, Bb, D), jnp.float32),
        scratch_types=[
            pltpu.VMEM((n_chunks, CHUNK), jnp.int32),
            pltpu.VMEM((nbuf, SUPER, D), jnp.float32),
        ]
        + [pltpu.SemaphoreType.DMA] * (2 * nbuf),
        compiler_params=pltpu.CompilerParams(use_tc_tiling_on_sc=False),
    )
    def gather_kernel(table_hbm, idx_hbm, out_hbm, idx_v, rows_v, *sems):
        gsem = sems[:nbuf]
        ssem = sems[nbuf:]
        wid = lax.axis_index("s") * NC + lax.axis_index("c")
        base = wid * b_per_w
        pltpu.sync_copy(idx_hbm.at[pl.ds(wid * n_chunks, n_chunks)], idx_v)

        def out_view(g, b):
            row = base + g * SUPER
            return out_hbm.at[row // Bb, pl.ds(row % Bb, SUPER)]

        def fire_gathers(g, b):
            for s in range(S):
                pltpu.async_copy(
                    table_hbm.at[idx_v.at[g * S + s]],
                    rows_v.at[b, pl.ds(s * CHUNK, CHUNK)],
                    gsem[b],
                )

        def wait_gathers(g, b):
            for s in range(S):
                pltpu.make_async_copy(
                    table_hbm.at[idx_v.at[g * S + s]],
                    rows_v.at[b, pl.ds(s * CHUNK, CHUNK)],
                    gsem[b],
                ).wait()

        def fire_store(g, b):
            pltpu.async_copy(rows_v.at[b], out_view(g, b), ssem[b])

        def wait_store(g, b):
            pltpu.make_async_copy(rows_v.at[b], out_view(g, b), ssem[b]).wait()

        for b in range(nbuf):
            fire_gathers(b, b)

        def outer_body(outer, carry):
            for db in range(nbuf):
                g = outer * nbuf + db
                bp = (db - 1) % nbuf
                gp = g - 1 + nbuf

                @pl.when((g >= 1) & (gp < n_super))
                def _():
                    wait_store(gp - nbuf, bp)
                    fire_gathers(gp, bp)

                wait_gathers(g, db)
                fire_store(g, db)
            return carry

        lax.fori_loop(0, n_super // nbuf, outer_body, 0)

        for b in range(nbuf):
            wait_store(n_super - nbuf + b, b)

    return gather_kernel


@functools.cache
def _make_tc_retile(H, Bb, D):
    BCHUNK = 1024
    n_j = Bb // BCHUNK
    zrows_per_blk = BCHUNK * D // 128

    def body(z_ref, out_ref):
        z = z_ref[...]
        a = z[:, :D]
        b = z[:, D:]
        at = jnp.transpose(a, (1, 0))
        bt = jnp.transpose(b, (1, 0))
        c = jnp.stack([at, bt], axis=-1).reshape(D, BCHUNK)
        out_ref[...] = c[None]

    return pl.pallas_call(
        body,
        grid=(H, n_j),
        in_specs=[
            pl.BlockSpec((zrows_per_blk, 128), lambda h, j: (h * n_j + j, 0))
        ],
        out_specs=pl.BlockSpec((1, D, BCHUNK), lambda h, j: (h, 0, j)),
        out_shape=jax.ShapeDtypeStruct((H, D, Bb), jnp.float32),
    )


def kernel(x, emb_table):
    Bb, H = x.shape
    D = emb_table.shape[1]
    idx = jnp.swapaxes(x, 0, 1).reshape(x.size // CHUNK, CHUNK).astype(jnp.int32)
    z = _make_gather(H, Bb, D)(emb_table, idx)
    z128 = z.reshape(x.size * D // 128, 128)
    r = _make_tc_retile(H, Bb, D)(z128)
    return jnp.transpose(r, (2, 0, 1))


# speedup vs baseline: 9.6927x; 9.6927x over previous
import functools

import jax
import jax.numpy as jnp
from jax import lax
from jax.experimental import pallas as pl
from jax.experimental.pallas import tpu as pltpu
from jax.experimental.pallas import tpu_sc as plsc

CHUNK = 128
S = 2
SUPER = S * CHUNK
NBUF = 4


@functools.cache
def _make_gather(H, Bb, D):
    B = H * Bb
    info = plsc.get_sparse_core_info()
    NC, NS = info.num_cores, info.num_subcores
    NW = NC * NS
    assert B % (NW * SUPER) == 0 and Bb % SUPER == 0
    b_per_w = B // NW
    n_chunks = b_per_w // CHUNK
    n_super = b_per_w // SUPER
    nbuf = max(n for n in (NBUF, 2, 1) if n_super % n == 0)

    mesh = plsc.VectorSubcoreMesh(core_axis_name="c", subcore_axis_name="s")

    @functools.partial(
        pl.kernel,
        mesh=mesh,
        out_type=jax.ShapeDtypeStruct((H, Bb, 128), jnp.float32),
        scratch_types=[
            pltpu.VMEM((n_chunks, CHUNK), jnp.int32),
            pltpu.VMEM((nbuf, SUPER, D), jnp.float32),
        ]
        + [pltpu.SemaphoreType.DMA] * (2 * nbuf),
        compiler_params=pltpu.CompilerParams(use_tc_tiling_on_sc=False),
    )
    def gather_kernel(table_hbm, idx_hbm, out_hbm, idx_v, rows_v, *sems):
        gsem = sems[:nbuf]
        ssem = sems[nbuf:]
        wid = lax.axis_index("s") * NC + lax.axis_index("c")
        base = wid * b_per_w
        pltpu.sync_copy(idx_hbm.at[pl.ds(wid * n_chunks, n_chunks)], idx_v)

        def out_view(g, b):
            row = base + g * SUPER
            return out_hbm.at[row // Bb, pl.ds(row % Bb, SUPER), pl.ds(0, D)]

        def fire_gathers(g, b):
            for s in range(S):
                pltpu.async_copy(
                    table_hbm.at[idx_v.at[g * S + s]],
                    rows_v.at[b, pl.ds(s * CHUNK, CHUNK)],
                    gsem[b],
                )

        def wait_gathers(g, b):
            for s in range(S):
                pltpu.make_async_copy(
                    table_hbm.at[idx_v.at[g * S + s]],
                    rows_v.at[b, pl.ds(s * CHUNK, CHUNK)],
                    gsem[b],
                ).wait()

        def fire_store(g, b):
            pltpu.async_copy(rows_v.at[b], out_view(g, b), ssem[b])

        def wait_store(g, b):
            pltpu.make_async_copy(rows_v.at[b], out_view(g, b), ssem[b]).wait()

        for b in range(nbuf):
            fire_gathers(b, b)

        def outer_body(outer, carry):
            for db in range(nbuf):
                g = outer * nbuf + db
                bp = (db - 1) % nbuf
                gp = g - 1 + nbuf

                @pl.when((g >= 1) & (gp < n_super))
                def _():
                    wait_store(gp - nbuf, bp)
                    fire_gathers(gp, bp)

                wait_gathers(g, db)
                fire_store(g, db)
            return carry

        lax.fori_loop(0, n_super // nbuf, outer_body, 0)

        for b in range(nbuf):
            wait_store(n_super - nbuf + b, b)

    return gather_kernel


@functools.cache
def _make_tc_retile(H, Bb, D):
    BCHUNK = 512
    n_j = Bb // BCHUNK

    def body(z_ref, out_ref):
        zt = jnp.transpose(z_ref[...], (1, 0))
        out_ref[...] = zt[None, :D, :]

    return pl.pallas_call(
        body,
        grid=(H, n_j),
        in_specs=[
            pl.BlockSpec((BCHUNK, 128), lambda h, j: (h * n_j + j, 0))
        ],
        out_specs=pl.BlockSpec((1, D, BCHUNK), lambda h, j: (h, 0, j)),
        out_shape=jax.ShapeDtypeStruct((H, D, Bb), jnp.float32),
    )


def kernel(x, emb_table):
    Bb, H = x.shape
    D = emb_table.shape[1]
    idx = jnp.swapaxes(x, 0, 1).reshape(x.size // CHUNK, CHUNK).astype(jnp.int32)
    zp = _make_gather(H, Bb, D)(emb_table, idx)
    z128 = zp.reshape(H * Bb, 128)
    r = _make_tc_retile(H, Bb, D)(z128)
    return jnp.transpose(r, (2, 0, 1))


# speedup vs baseline: 19.3226x; 1.9935x over previous
import functools

import jax
import jax.numpy as jnp
from jax import lax
from jax.experimental import pallas as pl
from jax.experimental.pallas import tpu as pltpu
from jax.experimental.pallas import tpu_sc as plsc

CHUNK = 128
S = 2
SUPER = S * CHUNK
NBUF = 4


@functools.cache
def _make_gather(H, Bb, D):
    B = H * Bb
    info = plsc.get_sparse_core_info()
    NC, NS = info.num_cores, info.num_subcores
    NW = NC * NS
    assert B % (NW * SUPER) == 0 and Bb % SUPER == 0
    b_per_w = B // NW
    n_chunks = b_per_w // CHUNK
    n_super = b_per_w // SUPER
    nbuf = max(n for n in (NBUF, 2, 1) if n_super % n == 0)

    mesh = plsc.VectorSubcoreMesh(core_axis_name="c", subcore_axis_name="s")

    @functools.partial(
        pl.kernel,
        mesh=mesh,
        out_type=jax.ShapeDtypeStruct((H, Bb, 128), jnp.float32),
        scratch_types=[
            pltpu.VMEM((n_chunks, CHUNK), jnp.int32),
            pltpu.VMEM((nbuf, SUPER, D), jnp.float32),
        ]
        + [pltpu.SemaphoreType.DMA] * (2 * nbuf),
        compiler_params=pltpu.CompilerParams(use_tc_tiling_on_sc=False),
    )
    def gather_kernel(table_hbm, idx_hbm, out_hbm, idx_v, rows_v, *sems):
        gsem = sems[:nbuf]
        ssem = sems[nbuf:]
        wid = lax.axis_index("s") * NC + lax.axis_index("c")
        base = wid * b_per_w
        pltpu.sync_copy(idx_hbm.at[pl.ds(wid * n_chunks, n_chunks)], idx_v)

        def out_view(g, b):
            row = base + g * SUPER
            return out_hbm.at[row // Bb, pl.ds(row % Bb, SUPER), pl.ds(0, D)]

        def fire_gathers(g, b):
            for s in range(S):
                pltpu.async_copy(
                    table_hbm.at[idx_v.at[g * S + s]],
                    rows_v.at[b, pl.ds(s * CHUNK, CHUNK)],
                    gsem[b],
                )

        def wait_gathers(g, b):
            for s in range(S):
                pltpu.make_async_copy(
                    table_hbm.at[idx_v.at[g * S + s]],
                    rows_v.at[b, pl.ds(s * CHUNK, CHUNK)],
                    gsem[b],
                ).wait()

        def fire_store(g, b):
            pltpu.async_copy(rows_v.at[b], out_view(g, b), ssem[b])

        def wait_store(g, b):
            pltpu.make_async_copy(rows_v.at[b], out_view(g, b), ssem[b]).wait()

        for b in range(nbuf):
            fire_gathers(b, b)

        def outer_body(outer, carry):
            for db in range(nbuf):
                g = outer * nbuf + db
                bp = (db - 1) % nbuf
                gp = g - 1 + nbuf

                @pl.when((g >= 1) & (gp < n_super))
                def _():
                    wait_store(gp - nbuf, bp)
                    fire_gathers(gp, bp)

                wait_gathers(g, db)
                fire_store(g, db)
            return carry

        lax.fori_loop(0, n_super // nbuf, outer_body, 0)

        for b in range(nbuf):
            wait_store(n_super - nbuf + b, b)

    return gather_kernel


@functools.cache
def _make_tc_retile(H, Bb, D):
    BCHUNK = 2048
    n_j = Bb // BCHUNK

    def body(z_ref, out_ref):
        zt = jnp.transpose(z_ref[...], (1, 0))
        out_ref[...] = zt[None, :D, :]

    return pl.pallas_call(
        body,
        grid=(H, n_j),
        in_specs=[
            pl.BlockSpec((BCHUNK, 128), lambda h, j: (h * n_j + j, 0))
        ],
        out_specs=pl.BlockSpec((1, D, BCHUNK), lambda h, j: (h, 0, j)),
        out_shape=jax.ShapeDtypeStruct((H, D, Bb), jnp.float32),
    )


def kernel(x, emb_table):
    Bb, H = x.shape
    D = emb_table.shape[1]
    idx = jnp.swapaxes(x, 0, 1).reshape(x.size // CHUNK, CHUNK).astype(jnp.int32)
    zp = _make_gather(H, Bb, D)(emb_table, idx)
    z128 = zp.reshape(H * Bb, 128)
    r = _make_tc_retile(H, Bb, D)(z128)
    return jnp.transpose(r, (2, 0, 1))


# speedup vs baseline: 26.3867x; 1.3656x over previous
import functools

import jax
import jax.numpy as jnp
from jax import lax
from jax.experimental import pallas as pl
from jax.experimental.pallas import tpu as pltpu
from jax.experimental.pallas import tpu_sc as plsc

CHUNK = 128
S = 2
SUPER = S * CHUNK
NBUF = 4


@functools.cache
def _make_gather(H, Bb, D):
    B = H * Bb
    info = plsc.get_sparse_core_info()
    NC, NS = info.num_cores, info.num_subcores
    NW = NC * NS
    assert B % (NW * SUPER) == 0 and Bb % SUPER == 0
    b_per_w = B // NW
    n_chunks = b_per_w // CHUNK
    n_super = b_per_w // SUPER
    nbuf = max(n for n in (NBUF, 2, 1) if n_super % n == 0)

    mesh = plsc.VectorSubcoreMesh(core_axis_name="c", subcore_axis_name="s")

    @functools.partial(
        pl.kernel,
        mesh=mesh,
        out_type=jax.ShapeDtypeStruct((H, Bb, 128), jnp.float32),
        scratch_types=[
            pltpu.VMEM((n_chunks, CHUNK), jnp.int32),
            pltpu.VMEM((nbuf, SUPER, D), jnp.float32),
        ]
        + [pltpu.SemaphoreType.DMA] * (2 * nbuf),
        compiler_params=pltpu.CompilerParams(use_tc_tiling_on_sc=False),
    )
    def gather_kernel(table_hbm, idx_hbm, out_hbm, idx_v, rows_v, *sems):
        gsem = sems[:nbuf]
        ssem = sems[nbuf:]
        wid = lax.axis_index("s") * NC + lax.axis_index("c")
        base = wid * b_per_w
        pltpu.sync_copy(idx_hbm.at[pl.ds(wid * n_chunks, n_chunks)], idx_v)

        def out_view(g, b):
            row = base + g * SUPER
            return out_hbm.at[row // Bb, pl.ds(row % Bb, SUPER), pl.ds(0, D)]

        def fire_gathers(g, b):
            for s in range(S):
                pltpu.async_copy(
                    table_hbm.at[idx_v.at[g * S + s]],
                    rows_v.at[b, pl.ds(s * CHUNK, CHUNK)],
                    gsem[b],
                )

        def wait_gathers(g, b):
            for s in range(S):
                pltpu.make_async_copy(
                    table_hbm.at[idx_v.at[g * S + s]],
                    rows_v.at[b, pl.ds(s * CHUNK, CHUNK)],
                    gsem[b],
                ).wait()

        def fire_store(g, b):
            pltpu.async_copy(rows_v.at[b], out_view(g, b), ssem[b])

        def wait_store(g, b):
            pltpu.make_async_copy(rows_v.at[b], out_view(g, b), ssem[b]).wait()

        for b in range(nbuf):
            fire_gathers(b, b)

        def outer_body(outer, carry):
            for db in range(nbuf):
                g = outer * nbuf + db
                bp = (db - 1) % nbuf
                gp = g - 1 + nbuf

                @pl.when((g >= 1) & (gp < n_super))
                def _():
                    wait_store(gp - nbuf, bp)
                    fire_gathers(gp, bp)

                wait_gathers(g, db)
                fire_store(g, db)
            return carry

        lax.fori_loop(0, n_super // nbuf, outer_body, 0)

        for b in range(nbuf):
            wait_store(n_super - nbuf + b, b)

    return gather_kernel


@functools.cache
def _make_tc_retile(H, Bb, D):
    BCHUNK = 8192
    n_j = Bb // BCHUNK

    def body(z_ref, out_ref):
        zt = jnp.transpose(z_ref[...], (1, 0))
        out_ref[...] = zt[None, :D, :]

    return pl.pallas_call(
        body,
        grid=(H, n_j),
        in_specs=[
            pl.BlockSpec((BCHUNK, 128), lambda h, j: (h * n_j + j, 0))
        ],
        out_specs=pl.BlockSpec((1, D, BCHUNK), lambda h, j: (h, 0, j)),
        out_shape=jax.ShapeDtypeStruct((H, D, Bb), jnp.float32),
    )


def kernel(x, emb_table):
    Bb, H = x.shape
    D = emb_table.shape[1]
    idx = jnp.swapaxes(x, 0, 1).reshape(x.size // CHUNK, CHUNK).astype(jnp.int32)
    zp = _make_gather(H, Bb, D)(emb_table, idx)
    z128 = zp.reshape(H * Bb, 128)
    r = _make_tc_retile(H, Bb, D)(z128)
    return jnp.transpose(r, (2, 0, 1))


# speedup vs baseline: 27.0574x; 1.0254x over previous
import functools

import jax
import jax.numpy as jnp
from jax import lax
from jax.experimental import pallas as pl
from jax.experimental.pallas import tpu as pltpu
from jax.experimental.pallas import tpu_sc as plsc

CHUNK = 128
S = 2
SUPER = S * CHUNK
NBUF = 4


@functools.cache
def _make_gather(H, Bb, D):
    B = H * Bb
    info = plsc.get_sparse_core_info()
    NC, NS = info.num_cores, info.num_subcores
    NW = NC * NS
    assert B % (NW * SUPER) == 0 and Bb % SUPER == 0
    b_per_w = B // NW
    n_chunks = b_per_w // CHUNK
    n_super = b_per_w // SUPER
    nbuf = max(n for n in (NBUF, 2, 1) if n_super % n == 0)

    mesh = plsc.VectorSubcoreMesh(core_axis_name="c", subcore_axis_name="s")

    @functools.partial(
        pl.kernel,
        mesh=mesh,
        out_type=jax.ShapeDtypeStruct((H, Bb, 128), jnp.float32),
        scratch_types=[
            pltpu.VMEM((n_chunks, CHUNK), jnp.int32),
            pltpu.VMEM((nbuf, SUPER, D), jnp.float32),
        ]
        + [pltpu.SemaphoreType.DMA] * (2 * nbuf),
        compiler_params=pltpu.CompilerParams(use_tc_tiling_on_sc=False),
    )
    def gather_kernel(table_hbm, idx_hbm, out_hbm, idx_v, rows_v, *sems):
        gsem = sems[:nbuf]
        ssem = sems[nbuf:]
        wid = lax.axis_index("s") * NC + lax.axis_index("c")
        base = wid * b_per_w
        pltpu.sync_copy(idx_hbm.at[pl.ds(wid * n_chunks, n_chunks)], idx_v)

        def out_view(g, b):
            row = base + g * SUPER
            return out_hbm.at[row // Bb, pl.ds(row % Bb, SUPER), pl.ds(0, D)]

        def fire_gathers(g, b):
            for s in range(S):
                pltpu.async_copy(
                    table_hbm.at[idx_v.at[g * S + s]],
                    rows_v.at[b, pl.ds(s * CHUNK, CHUNK)],
                    gsem[b],
                )

        def wait_gathers(g, b):
            for s in range(S):
                pltpu.make_async_copy(
                    table_hbm.at[idx_v.at[g * S + s]],
                    rows_v.at[b, pl.ds(s * CHUNK, CHUNK)],
                    gsem[b],
                ).wait()

        def fire_store(g, b):
            pltpu.async_copy(rows_v.at[b], out_view(g, b), ssem[b])

        def wait_store(g, b):
            pltpu.make_async_copy(rows_v.at[b], out_view(g, b), ssem[b]).wait()

        for b in range(nbuf):
            fire_gathers(b, b)

        def outer_body(outer, carry):
            for db in range(nbuf):
                g = outer * nbuf + db
                bp = (db - 1) % nbuf
                gp = g - 1 + nbuf

                @pl.when((g >= 1) & (gp < n_super))
                def _():
                    wait_store(gp - nbuf, bp)
                    fire_gathers(gp, bp)

                wait_gathers(g, db)
                fire_store(g, db)
            return carry

        lax.fori_loop(0, n_super // nbuf, outer_body, 0)

        for b in range(nbuf):
            wait_store(n_super - nbuf + b, b)

    return gather_kernel


@functools.cache
def _make_tc_retile(H, Bb, D):
    BCHUNK = 16384
    n_j = Bb // BCHUNK

    def body(z_ref, out_ref):
        zt = jnp.transpose(z_ref[...], (1, 0))
        out_ref[...] = zt[None, :D, :]

    return pl.pallas_call(
        body,
        grid=(H, n_j),
        in_specs=[
            pl.BlockSpec((BCHUNK, 128), lambda h, j: (h * n_j + j, 0))
        ],
        out_specs=pl.BlockSpec((1, D, BCHUNK), lambda h, j: (h, 0, j)),
        out_shape=jax.ShapeDtypeStruct((H, D, Bb), jnp.float32),
    )


def kernel(x, emb_table):
    Bb, H = x.shape
    D = emb_table.shape[1]
    idx = jnp.swapaxes(x, 0, 1).reshape(x.size // CHUNK, CHUNK).astype(jnp.int32)
    zp = _make_gather(H, Bb, D)(emb_table, idx)
    z128 = zp.reshape(H * Bb, 128)
    r = _make_tc_retile(H, Bb, D)(z128)
    return jnp.transpose(r, (2, 0, 1))
